# baseline (speedup 1.0000x reference)
import jax
import jax.numpy as jnp
from jax.experimental import pallas as pl

_E = 64
_K = 8


def _eye(n):
    r = jax.lax.broadcasted_iota(jnp.int32, (n, n), 0)
    c = jax.lax.broadcasted_iota(jnp.int32, (n, n), 1)
    return (r == c).astype(jnp.float32)


def _router_kernel(x_ref, w_ref, scores_ref, ew_ref, ei_ref):
    x = x_ref[...]
    w = w_ref[...]
    lt = jax.lax.dot_general(
        w, x, (((1,), (1,)), ((), ())), preferred_element_type=jnp.float32
    )
    m = jnp.max(lt, axis=0, keepdims=True)
    e = jnp.exp(lt - m)
    s = e / jnp.sum(e, axis=0, keepdims=True)
    scores_ref[...] = jax.lax.dot_general(
        s, _eye(_E), (((0,), (0,)), ((), ())),
        preferred_element_type=jnp.float32,
        precision=jax.lax.Precision.HIGHEST,
    )

    iota = jax.lax.broadcasted_iota(jnp.int32, s.shape, 0)
    val = s
    ew_rows = []
    ei_rows = []
    for _ in range(_K):
        mx = jnp.max(val, axis=0, keepdims=True)
        idx = jnp.min(jnp.where(val == mx, iota, _E), axis=0, keepdims=True)
        ew_rows.append(mx)
        ei_rows.append(idx)
        val = jnp.where(iota == idx, -1.0, val)
    ewt = jnp.concatenate(ew_rows, axis=0)
    eit = jnp.concatenate(ei_rows, axis=0).astype(jnp.float32)
    eye_k = _eye(_K)
    ew_ref[...] = jax.lax.dot_general(
        ewt, eye_k, (((0,), (0,)), ((), ())),
        preferred_element_type=jnp.float32,
        precision=jax.lax.Precision.HIGHEST,
    )
    ei_ref[...] = jax.lax.dot_general(
        eit, eye_k, (((0,), (0,)), ((), ())),
        preferred_element_type=jnp.float32,
    ).astype(jnp.int32)


def kernel(x, W):
    sl, bs, hs = x.shape
    m = sl * bs
    x2 = x.reshape(m, hs)
    tm = 256
    scores, ew, ei = pl.pallas_call(
        _router_kernel,
        grid=(m // tm,),
        in_specs=[
            pl.BlockSpec((tm, hs), lambda i: (i, 0)),
            pl.BlockSpec((_E, hs), lambda i: (0, 0)),
        ],
        out_specs=[
            pl.BlockSpec((tm, _E), lambda i: (i, 0)),
            pl.BlockSpec((tm, _K), lambda i: (i, 0)),
            pl.BlockSpec((tm, _K), lambda i: (i, 0)),
        ],
        out_shape=[
            jax.ShapeDtypeStruct((m, _E), jnp.float32),
            jax.ShapeDtypeStruct((m, _K), jnp.float32),
            jax.ShapeDtypeStruct((m, _K), jnp.int32),
        ],
    )(x2, W)
    return scores, ew, ei, jnp.float32(0.0)
